# baseline (speedup 1.0000x reference)
import functools

import jax
import jax.numpy as jnp
from jax import lax
from jax.experimental import pallas as pl
from jax.experimental.pallas import tpu as pltpu
from jax.experimental.pallas import tpu_sc as plsc

N = 10000
E = 320000
D = 128

NC = 2
NS = 16
CHUNK = 128
CPT = 79
E_PAD = NC * NS * CPT * CHUNK
BS = 56
NPA, NPB = 28, 12
K0 = 2 * (NPA + NPB)
K1 = 2 * (NPA + NPB)
BSTG = BS + 1
CPTA = BS + BSTG
N_PAD = 10240
RPT = N_PAD // NS
CNTW = 128


def _scatter_body(src_hbm, dst_hbm, feat_hbm, zacc_hbm,
                  acc_out, src_v, dst_v, rows_v, acc_sh, sem):
    c = lax.axis_index("c")
    s = lax.axis_index("s")
    row0 = s * RPT
    pltpu.sync_copy(zacc_hbm.at[pl.ds(row0, RPT)], acc_sh.at[pl.ds(row0, RPT)])
    pltpu.sync_copy(src_hbm.at[c, s], src_v)
    pltpu.sync_copy(dst_hbm.at[c, s], dst_v)
    plsc.subcore_barrier()

    nblocks = 2
    rows0 = rows_v.at[0]
    rows1 = rows_v.at[1]

    def block(b, carry):
        npairs = jnp.where(b == 0, NPA, NPB)
        w = c * NS + s
        pltpu.sync_copy(src_hbm.at[w, b], src_v)
        pltpu.sync_copy(dst_hbm.at[w, b], dst_v)
        pltpu.async_copy(feat_hbm.at[src_v.at[0]], rows0, sem)

        def step(i, carry2):
            j0 = 2 * i
            pltpu.make_async_copy(feat_hbm.at[src_v.at[j0]], rows0, sem).wait()
            pltpu.async_copy(feat_hbm.at[src_v.at[j0 + 1]], rows1, sem)
            pltpu.sync_copy(rows0, acc_sh.at[dst_v.at[j0]], add=True)
            pltpu.make_async_copy(feat_hbm.at[src_v.at[j0 + 1]], rows1,
                                  sem).wait()
            pltpu.async_copy(feat_hbm.at[src_v.at[j0 + 2]], rows0, sem)
            pltpu.sync_copy(rows1, acc_sh.at[dst_v.at[j0 + 1]], add=True)
            return carry2

        lax.fori_loop(0, npairs, step, 0)
        pltpu.make_async_copy(feat_hbm.at[src_v.at[0]], rows0, sem).wait()
        return carry

    lax.fori_loop(0, nblocks, block, 0)
    plsc.subcore_barrier()
    pltpu.sync_copy(acc_sh.at[pl.ds(row0, RPT)],
                    acc_out.at[c, pl.ds(row0, RPT)])


def _make_scatter():
    mesh = plsc.VectorSubcoreMesh(core_axis_name="c", subcore_axis_name="s")
    return pl.kernel(
        _scatter_body,
        out_type=jax.ShapeDtypeStruct((NC, N_PAD, D), jnp.float32),
        mesh=mesh,
        scratch_types=[
            pltpu.VMEM((BSTG, CHUNK), jnp.int32),
            pltpu.VMEM((BSTG, CHUNK), jnp.int32),
            pltpu.VMEM((2, CHUNK, D), jnp.float32),
            pltpu.VMEM_SHARED((N_PAD, D), jnp.float32),
            pltpu.SemaphoreType.DMA,
        ],
    )


def _cnt_body(dst_hbm, zcnt_hbm, ones_hbm,
              cnt_out, dst_v, ones_v, cnt_sh):
    c = lax.axis_index("c")
    s = lax.axis_index("s")
    row0 = s * RPT
    pltpu.sync_copy(zcnt_hbm.at[pl.ds(row0, RPT)], cnt_sh.at[pl.ds(row0, RPT)])
    pltpu.sync_copy(dst_hbm.at[c, s], dst_v)
    pltpu.sync_copy(ones_hbm, ones_v)
    plsc.subcore_barrier()

    def step(j, carry):
        pltpu.sync_copy(ones_v, cnt_sh.at[dst_v.at[j]], add=True)
        return carry

    lax.fori_loop(0, CPT, step, 0)
    plsc.subcore_barrier()
    pltpu.sync_copy(cnt_sh.at[pl.ds(row0, RPT)],
                    cnt_out.at[c, pl.ds(row0, RPT)])


def _make_cnt():
    mesh = plsc.VectorSubcoreMesh(core_axis_name="c", subcore_axis_name="s")
    return pl.kernel(
        _cnt_body,
        out_type=jax.ShapeDtypeStruct((NC, N_PAD, CNTW), jnp.float32),
        mesh=mesh,
        scratch_types=[
            pltpu.VMEM((CPT, CHUNK), jnp.int32),
            pltpu.VMEM((CHUNK, CNTW), jnp.float32),
            pltpu.VMEM_SHARED((N_PAD, CNTW), jnp.float32),
        ],
    )


def _asym_build(flat, fill):
    n0 = NS * K0 * CHUNK
    n1 = NS * K1 * CHUNK
    p1 = jnp.concatenate(
        [flat[n0:], jnp.full((n0 + n1 - flat.shape[0],), fill, jnp.int32)])
    parts = []
    for p, k in ((flat[:n0].reshape(NS, K0, CHUNK), K0),
                 (p1.reshape(NS, K1, CHUNK), K1)):
        parts.append(jnp.concatenate(
            [p, jnp.full((NS, CPTA - k, CHUNK), fill, jnp.int32)], axis=1))
    a = jnp.stack(parts)
    w = jnp.stack([a[:, :, :BSTG], a[:, :, BS:BS + BSTG]], axis=2)
    return w.reshape(NC * NS, 2, BSTG, CHUNK)


def _clip_body(x_ref, o_ref):
    o_ref[...] = jnp.clip(x_ref[...], -1000.0, 1000.0)


def _layer_body(accp_ref, cntp_ref, x_ref, wl_ref, wr_ref, bl_ref, o_ref):
    a = accp_ref[...]
    agg = a[0] + a[1]
    cn = cntp_ref[...]
    cnt = cn[0, :, 0:1] + cn[1, :, 0:1]
    mean = agg / jnp.maximum(cnt, 1.0)
    h = lax.dot_general(mean, wl_ref[...], (((1,), (1,)), ((), ())),
                        preferred_element_type=jnp.float32)
    h = h + lax.dot_general(x_ref[...], wr_ref[...], (((1,), (1,)), ((), ())),
                            preferred_element_type=jnp.float32)
    h = h + bl_ref[...]
    o_ref[...] = jnp.minimum(jnp.maximum(h, 0.0), 1000.0)


_ROWS_BLK = 1000
_GRID = N // _ROWS_BLK


def _layer_call(accp, cntp, xin, Wl, bl, Wr):
    return pl.pallas_call(
        _layer_body,
        grid=(_GRID,),
        in_specs=[
            pl.BlockSpec((NC, _ROWS_BLK, D), lambda i: (0, i, 0)),
            pl.BlockSpec((NC, _ROWS_BLK, CNTW), lambda i: (0, i, 0)),
            pl.BlockSpec((_ROWS_BLK, D), lambda i: (i, 0)),
            pl.BlockSpec((D, D), lambda i: (0, 0)),
            pl.BlockSpec((D, D), lambda i: (0, 0)),
            pl.BlockSpec((1, D), lambda i: (0, 0)),
        ],
        out_specs=pl.BlockSpec((_ROWS_BLK, D), lambda i: (i, 0)),
        out_shape=jax.ShapeDtypeStruct((N, D), jnp.float32),
    )(accp, cntp, xin, Wl, Wr, bl.reshape(1, D))


def kernel(x, edge_index, Wl1, bl1, Wr1, Wl2, bl2, Wr2):
    ei = edge_index.astype(jnp.int32)
    pad = E_PAD - E
    src = jnp.concatenate([ei[0], jnp.zeros((pad,), jnp.int32)])
    dst = jnp.concatenate([ei[1], jnp.full((pad,), N_PAD - 1, jnp.int32)])
    dst_r = dst.reshape(NC, NS, CPT, CHUNK)

    src_a = _asym_build(src[:E], 0)
    dst_a = _asym_build(dst[:E], N_PAD - 1)
    zacc = jnp.zeros((N_PAD, D), jnp.float32)
    zcnt = jnp.zeros((N_PAD, CNTW), jnp.float32)
    ones = jnp.ones((CHUNK, CNTW), jnp.float32)

    xc = pl.pallas_call(
        _clip_body,
        grid=(_GRID,),
        in_specs=[pl.BlockSpec((_ROWS_BLK, D), lambda i: (i, 0))],
        out_specs=pl.BlockSpec((_ROWS_BLK, D), lambda i: (i, 0)),
        out_shape=jax.ShapeDtypeStruct((N, D), jnp.float32),
    )(x)

    acc1 = _make_scatter()(src_a, dst_a, xc, zacc)
    cnt1 = _make_cnt()(dst_r, zcnt, ones)
    h1 = _layer_call(acc1, cnt1, xc, Wl1, bl1, Wr1)
    acc2 = _make_scatter()(src_a, dst_a, h1, zacc)
    return _layer_call(acc2, cnt1, h1, Wl2, bl2, Wr2)


# speedup vs baseline: 1.9290x; 1.9290x over previous
import functools

import jax
import jax.numpy as jnp
from jax import lax
from jax.experimental import pallas as pl
from jax.experimental.pallas import tpu as pltpu
from jax.experimental.pallas import tpu_sc as plsc

N = 10000
E = 320000
D = 128

NC = 2
NS = 16
CHUNK = 128
CPT = 79
E_PAD = NC * NS * CPT * CHUNK

N_PAD = 10240
RPT = N_PAD // NS
CNTW = 128


def _scatter_body(src_hbm, dst_hbm, feat_hbm, zacc_hbm,
                  acc_out, src_v, dst_v, rows_v, acc_sh, sem):
    c = lax.axis_index("c")
    s = lax.axis_index("s")
    row0 = s * RPT
    pltpu.sync_copy(zacc_hbm.at[pl.ds(row0, RPT)], acc_sh.at[pl.ds(row0, RPT)])
    pltpu.sync_copy(src_hbm.at[c, s], src_v)
    pltpu.sync_copy(dst_hbm.at[c, s], dst_v)
    plsc.subcore_barrier()

    def step(j, carry):
        pltpu.async_copy(feat_hbm.at[src_v.at[j]], rows_v, sem).wait()
        pltpu.sync_copy(rows_v, acc_sh.at[dst_v.at[j]], add=True)
        return carry

    lax.fori_loop(0, CPT, step, 0)
    plsc.subcore_barrier()
    pltpu.sync_copy(acc_sh.at[pl.ds(row0, RPT)],
                    acc_out.at[c, pl.ds(row0, RPT)])


def _make_scatter():
    mesh = plsc.VectorSubcoreMesh(core_axis_name="c", subcore_axis_name="s")
    return pl.kernel(
        _scatter_body,
        out_type=jax.ShapeDtypeStruct((NC, N_PAD, D), jnp.float32),
        mesh=mesh,
        scratch_types=[
            pltpu.VMEM((CPT, CHUNK), jnp.int32),
            pltpu.VMEM((CPT, CHUNK), jnp.int32),
            pltpu.VMEM((CHUNK, D), jnp.float32),
            pltpu.VMEM_SHARED((N_PAD, D), jnp.float32),
            pltpu.SemaphoreType.DMA,
        ],
    )


def _cnt_body(dst_hbm, zcnt_hbm, ones_hbm,
              cnt_out, dst_v, ones_v, cnt_sh):
    c = lax.axis_index("c")
    s = lax.axis_index("s")
    row0 = s * RPT
    pltpu.sync_copy(zcnt_hbm.at[pl.ds(row0, RPT)], cnt_sh.at[pl.ds(row0, RPT)])
    pltpu.sync_copy(dst_hbm.at[c, s], dst_v)
    pltpu.sync_copy(ones_hbm, ones_v)
    plsc.subcore_barrier()

    def step(j, carry):
        pltpu.sync_copy(ones_v, cnt_sh.at[dst_v.at[j]], add=True)
        return carry

    lax.fori_loop(0, CPT, step, 0)
    plsc.subcore_barrier()
    pltpu.sync_copy(cnt_sh.at[pl.ds(row0, RPT)],
                    cnt_out.at[c, pl.ds(row0, RPT)])


def _make_cnt():
    mesh = plsc.VectorSubcoreMesh(core_axis_name="c", subcore_axis_name="s")
    return pl.kernel(
        _cnt_body,
        out_type=jax.ShapeDtypeStruct((NC, N_PAD, CNTW), jnp.float32),
        mesh=mesh,
        scratch_types=[
            pltpu.VMEM((CPT, CHUNK), jnp.int32),
            pltpu.VMEM((CHUNK, CNTW), jnp.float32),
            pltpu.VMEM_SHARED((N_PAD, CNTW), jnp.float32),
        ],
    )


def _clip_body(x_ref, o_ref):
    o_ref[...] = jnp.clip(x_ref[...], -1000.0, 1000.0)


def _layer_body(accp_ref, cntp_ref, x_ref, wl_ref, wr_ref, bl_ref, o_ref):
    a = accp_ref[...]
    agg = a[0] + a[1]
    cn = cntp_ref[...]
    cnt = cn[0, :, 0:1] + cn[1, :, 0:1]
    mean = agg / jnp.maximum(cnt, 1.0)
    h = lax.dot_general(mean, wl_ref[...], (((1,), (1,)), ((), ())),
                        preferred_element_type=jnp.float32)
    h = h + lax.dot_general(x_ref[...], wr_ref[...], (((1,), (1,)), ((), ())),
                            preferred_element_type=jnp.float32)
    h = h + bl_ref[...]
    o_ref[...] = jnp.minimum(jnp.maximum(h, 0.0), 1000.0)


_ROWS_BLK = 1000
_GRID = N // _ROWS_BLK


def _layer_call(accp, cntp, xin, Wl, bl, Wr):
    return pl.pallas_call(
        _layer_body,
        grid=(_GRID,),
        in_specs=[
            pl.BlockSpec((NC, _ROWS_BLK, D), lambda i: (0, i, 0)),
            pl.BlockSpec((NC, _ROWS_BLK, CNTW), lambda i: (0, i, 0)),
            pl.BlockSpec((_ROWS_BLK, D), lambda i: (i, 0)),
            pl.BlockSpec((D, D), lambda i: (0, 0)),
            pl.BlockSpec((D, D), lambda i: (0, 0)),
            pl.BlockSpec((1, D), lambda i: (0, 0)),
        ],
        out_specs=pl.BlockSpec((_ROWS_BLK, D), lambda i: (i, 0)),
        out_shape=jax.ShapeDtypeStruct((N, D), jnp.float32),
    )(accp, cntp, xin, Wl, Wr, bl.reshape(1, D))


def kernel(x, edge_index, Wl1, bl1, Wr1, Wl2, bl2, Wr2):
    ei = edge_index.astype(jnp.int32)
    pad = E_PAD - E
    pad_dst = N + (jnp.arange(pad, dtype=jnp.int32) % (N_PAD - N))
    src = jnp.concatenate([ei[0], jnp.zeros((pad,), jnp.int32)])
    dst = jnp.concatenate([ei[1], pad_dst])
    src_r = src.reshape(NC, NS, CPT, CHUNK)
    dst_r = dst.reshape(NC, NS, CPT, CHUNK)
    zacc = jnp.zeros((N_PAD, D), jnp.float32)
    zcnt = jnp.zeros((N_PAD, CNTW), jnp.float32)
    ones = jnp.ones((CHUNK, CNTW), jnp.float32)

    xc = pl.pallas_call(
        _clip_body,
        grid=(_GRID,),
        in_specs=[pl.BlockSpec((_ROWS_BLK, D), lambda i: (i, 0))],
        out_specs=pl.BlockSpec((_ROWS_BLK, D), lambda i: (i, 0)),
        out_shape=jax.ShapeDtypeStruct((N, D), jnp.float32),
    )(x)

    acc1 = _make_scatter()(src_r, dst_r, xc, zacc)
    cnt1 = _make_cnt()(dst_r, zcnt, ones)
    h1 = _layer_call(acc1, cnt1, xc, Wl1, bl1, Wr1)
    acc2 = _make_scatter()(src_r, dst_r, h1, zacc)
    return _layer_call(acc2, cnt1, h1, Wl2, bl2, Wr2)
